# baseline (speedup 1.0000x reference)
import functools

import jax
import jax.numpy as jnp
from jax import lax
from jax.experimental import pallas as pl
from jax.experimental.pallas import tpu as pltpu
from jax.experimental.pallas import tpu_sc as plsc

N_NODES = 50000
N_PAD = 50048
B_STATIC = 4
E_STATIC = 1600000
VARPHI = 0.7
NUM_CORES = 2
NUM_SUBCORES = 16
NW = NUM_CORES * NUM_SUBCORES
GROUPS_PER_BATCH = 8
EDGES_PER_TILE = E_STATIC // GROUPS_PER_BATCH
CH = 1600
NCHUNK = EDGES_PER_TILE // CH
WCH = 1792
K_MEDIAN = (N_NODES - 1) // 2


def _sc_degree_partials(eidx_lin, weights):
    mesh = plsc.VectorSubcoreMesh(core_axis_name="c", subcore_axis_name="s")

    @functools.partial(
        pl.kernel,
        mesh=mesh,
        out_type=jax.ShapeDtypeStruct((NW * N_PAD,), jnp.float32),
        scratch_types=[
            pltpu.VMEM((N_PAD,), jnp.float32),
            pltpu.VMEM((N_PAD,), jnp.float32),
            pltpu.VMEM((CH,), jnp.int32),
            pltpu.VMEM((CH,), jnp.int32),
            pltpu.VMEM((CH,), jnp.int32),
            pltpu.VMEM((CH,), jnp.int32),
            pltpu.VMEM((B_STATIC, WCH), jnp.float32),
            pltpu.VMEM((B_STATIC, WCH), jnp.float32),
            pltpu.SemaphoreType.DMA,
            pltpu.SemaphoreType.DMA,
            pltpu.SemaphoreType.DMA,
            pltpu.SemaphoreType.DMA,
            pltpu.SemaphoreType.DMA,
            pltpu.SemaphoreType.DMA,
        ],
        compiler_params=pltpu.CompilerParams(needs_layout_passes=False),
    )
    def sc_kernel(eidx_hbm, w_hbm, out_hbm, dega, degb, sb0, sb1, db0, db1,
                  wb0, wb1, ss0, ss1, sd0, sd1, sw0, sw1):
        wid = lax.axis_index("s") * NUM_CORES + lax.axis_index("c")
        b = lax.shift_right_logical(wid, 3)
        g = lax.bitwise_and(wid, 7)
        e_base = g * EDGES_PER_TILE
        src_base = b * (2 * E_STATIC) + e_base
        dst_base = src_base + E_STATIC
        sbufs = (sb0, sb1)
        dbufs = (db0, db1)
        wbufs = (wb0, wb1)
        ssems = (ss0, ss1)
        dsems = (sd0, sd1)
        wsems = (sw0, sw1)
        zeros16 = jnp.zeros((16,), jnp.float32)

        def zero_body(i, carry):
            dega[pl.ds(i * 16, 16)] = zeros16
            degb[pl.ds(i * 16, 16)] = zeros16
            return carry
        lax.fori_loop(0, N_PAD // 16, zero_body, 0)

        def fire(c, i):
            off = c * CH
            e0 = e_base + off
            wa = pl.multiple_of(lax.bitwise_and(e0, ~127), 128)
            pltpu.async_copy(eidx_hbm.at[pl.ds(src_base + off, CH)],
                             sbufs[i], ssems[i])
            pltpu.async_copy(eidx_hbm.at[pl.ds(dst_base + off, CH)],
                             dbufs[i], dsems[i])
            pltpu.async_copy(w_hbm.at[:, pl.ds(wa, WCH)],
                             wbufs[i], wsems[i])

        def wait(i):
            pltpu.make_async_copy(eidx_hbm.at[pl.ds(0, CH)],
                                  sbufs[i], ssems[i]).wait()
            pltpu.make_async_copy(eidx_hbm.at[pl.ds(0, CH)],
                                  dbufs[i], dsems[i]).wait()
            pltpu.make_async_copy(w_hbm.at[:, pl.ds(0, WCH)],
                                  wbufs[i], wsems[i]).wait()

        def process(c, i):
            sb = sbufs[i]
            db = dbufs[i]
            wb = wbufs[i]
            woff = lax.bitwise_and(e_base + c * CH, 127)

            def grp_body(q, carry):
                o = q * 80
                for u in range(5):
                    ou = o + u * 16
                    sv = sb[pl.ds(ou, 16)]
                    dv = db[pl.ds(ou, 16)]
                    wv = wb[b, pl.ds(woff + ou, 16)]
                    plsc.addupdate_scatter(dega, [sv], wv)
                    plsc.addupdate_scatter(degb, [dv], wv)
                return carry
            lax.fori_loop(0, CH // 80, grp_body, 0)

        fire(0, 0)

        def pipe_body(t, carry):
            c0 = 2 * t
            c1 = 2 * t + 1
            fire(c1, 1)
            wait(0)
            process(c0, 0)

            @pl.when(c0 + 2 < NCHUNK)
            def _():
                fire(c0 + 2, 0)

            wait(1)
            process(c1, 1)
            return carry
        lax.fori_loop(0, NCHUNK // 2, pipe_body, 0)
        if NCHUNK % 2 == 1:
            wait(0)
            process(NCHUNK - 1, 0)

        def merge_body(i, carry):
            o = i * 16
            dega[pl.ds(o, 16)] = dega[pl.ds(o, 16)] + degb[pl.ds(o, 16)]
            return carry
        lax.fori_loop(0, N_PAD // 16, merge_body, 0)

        pltpu.sync_copy(dega, out_hbm.at[pl.ds(wid * N_PAD, N_PAD)])

    return sc_kernel(eidx_lin, weights)


def _tc_threshold(partials):
    def body(p_ref, o_ref):
        deg = p_ref[:, 0, :]
        for w in range(1, GROUPS_PER_BATCH):
            deg = deg + p_ref[:, w, :]
        rmax = jnp.max(deg, axis=1, keepdims=True)
        xu = lax.bitcast_convert_type(deg, jnp.uint32)
        col = lax.broadcasted_iota(jnp.int32, (B_STATIC, N_PAD), 1)
        valid = col < N_NODES

        def radix_body(i, p):
            bit = (31 - i).astype(jnp.uint32)
            t = p | (jnp.uint32(1) << bit)
            hit = jnp.logical_and(xu < t, valid)
            cnt = jnp.sum(hit.astype(jnp.int32), axis=1, keepdims=True)
            return jnp.where(cnt <= K_MEDIAN, t, p)

        p = lax.fori_loop(0, 32, radix_body,
                          jnp.zeros((B_STATIC, 1), jnp.uint32))
        kth = lax.bitcast_convert_type(p, jnp.float32)
        denom = rmax + 1e-8
        norm = deg / denom
        thr = kth / denom
        phi = jnp.where(norm >= thr, jnp.float32(1.0), jnp.float32(VARPHI))
        o_ref[...] = phi[:, :N_NODES]

    return pl.pallas_call(
        body,
        out_shape=jax.ShapeDtypeStruct((B_STATIC, N_NODES), jnp.float32),
    )(partials)


def kernel(edge_index, edge_weights, num_nodes):
    eidx_lin = edge_index.transpose(0, 2, 1).reshape(-1)
    partials = _sc_degree_partials(eidx_lin, edge_weights)
    partials = partials.reshape(B_STATIC, GROUPS_PER_BATCH, N_PAD)
    return _tc_threshold(partials)


# speedup vs baseline: 1.2207x; 1.2207x over previous
import functools

import jax
import jax.numpy as jnp
from jax import lax
from jax.experimental import pallas as pl
from jax.experimental.pallas import tpu as pltpu
from jax.experimental.pallas import tpu_sc as plsc

N_NODES = 50000
N_PAD = 50048
B_STATIC = 4
E_STATIC = 1600000
VARPHI = 0.7
NUM_CORES = 2
NUM_SUBCORES = 16
NW = NUM_CORES * NUM_SUBCORES
GROUPS_PER_BATCH = 8
EDGES_PER_TILE = E_STATIC // GROUPS_PER_BATCH
CH = 4000
NCHUNK = EDGES_PER_TILE // CH
WCH = 4224
K_MEDIAN = (N_NODES - 1) // 2


def _sc_degree_partials(eidx_lin, weights):
    mesh = plsc.VectorSubcoreMesh(core_axis_name="c", subcore_axis_name="s")

    @functools.partial(
        pl.kernel,
        mesh=mesh,
        out_type=jax.ShapeDtypeStruct((NW * N_PAD,), jnp.float32),
        scratch_types=[
            pltpu.VMEM((N_PAD,), jnp.float32),
            pltpu.VMEM((CH,), jnp.int32),
            pltpu.VMEM((CH,), jnp.int32),
            pltpu.VMEM((CH,), jnp.int32),
            pltpu.VMEM((CH,), jnp.int32),
            pltpu.VMEM((B_STATIC, WCH), jnp.float32),
            pltpu.VMEM((B_STATIC, WCH), jnp.float32),
            pltpu.SemaphoreType.DMA,
            pltpu.SemaphoreType.DMA,
            pltpu.SemaphoreType.DMA,
            pltpu.SemaphoreType.DMA,
            pltpu.SemaphoreType.DMA,
            pltpu.SemaphoreType.DMA,
        ],
        compiler_params=pltpu.CompilerParams(needs_layout_passes=False),
    )
    def sc_kernel(eidx_hbm, w_hbm, out_hbm, deg, sb0, sb1, db0, db1,
                  wb0, wb1, ss0, ss1, sd0, sd1, sw0, sw1):
        wid = lax.axis_index("s") * NUM_CORES + lax.axis_index("c")
        b = lax.shift_right_logical(wid, 3)
        g = lax.bitwise_and(wid, 7)
        e_base = g * EDGES_PER_TILE
        src_base = b * (2 * E_STATIC) + e_base
        dst_base = src_base + E_STATIC
        sbufs = (sb0, sb1)
        dbufs = (db0, db1)
        wbufs = (wb0, wb1)
        ssems = (ss0, ss1)
        dsems = (sd0, sd1)
        wsems = (sw0, sw1)
        zeros16 = jnp.zeros((16,), jnp.float32)

        def zero_body(i, carry):
            deg[pl.ds(i * 16, 16)] = zeros16
            return carry
        lax.fori_loop(0, N_PAD // 16, zero_body, 0)

        def fire(c, i):
            off = c * CH
            e0 = e_base + off
            wa = pl.multiple_of(lax.bitwise_and(e0, ~127), 128)
            pltpu.async_copy(eidx_hbm.at[pl.ds(src_base + off, CH)],
                             sbufs[i], ssems[i])
            pltpu.async_copy(eidx_hbm.at[pl.ds(dst_base + off, CH)],
                             dbufs[i], dsems[i])
            pltpu.async_copy(w_hbm.at[:, pl.ds(wa, WCH)],
                             wbufs[i], wsems[i])

        def wait(i):
            pltpu.make_async_copy(eidx_hbm.at[pl.ds(0, CH)],
                                  sbufs[i], ssems[i]).wait()
            pltpu.make_async_copy(eidx_hbm.at[pl.ds(0, CH)],
                                  dbufs[i], dsems[i]).wait()
            pltpu.make_async_copy(w_hbm.at[:, pl.ds(0, WCH)],
                                  wbufs[i], wsems[i]).wait()

        def process(c, i):
            sb = sbufs[i]
            db = dbufs[i]
            wb = wbufs[i]
            woff = lax.bitwise_and(e_base + c * CH, 127)

            @functools.partial(plsc.parallel_loop, 0, CH // 16, unroll=8)
            def grp_body(q):
                ou = q * 16
                sv = sb[pl.ds(ou, 16)]
                dv = db[pl.ds(ou, 16)]
                wv = wb[b, pl.ds(woff + ou, 16)]
                plsc.addupdate_scatter(deg, [sv], wv)
                plsc.addupdate_scatter(deg, [dv], wv)

        fire(0, 0)

        def pipe_body(t, carry):
            c0 = 2 * t
            c1 = 2 * t + 1
            fire(c1, 1)
            wait(0)
            process(c0, 0)

            @pl.when(c0 + 2 < NCHUNK)
            def _():
                fire(c0 + 2, 0)

            wait(1)
            process(c1, 1)
            return carry
        lax.fori_loop(0, NCHUNK // 2, pipe_body, 0)
        if NCHUNK % 2 == 1:
            wait(0)
            process(NCHUNK - 1, 0)

        pltpu.sync_copy(deg, out_hbm.at[pl.ds(wid * N_PAD, N_PAD)])

    return sc_kernel(eidx_lin, weights)


def _tc_threshold(partials):
    def body(p_ref, o_ref):
        deg = p_ref[:, 0, :]
        for w in range(1, GROUPS_PER_BATCH):
            deg = deg + p_ref[:, w, :]
        rmax = jnp.max(deg, axis=1, keepdims=True)
        xu = lax.bitcast_convert_type(deg, jnp.uint32)
        col = lax.broadcasted_iota(jnp.int32, (B_STATIC, N_PAD), 1)
        valid = col < N_NODES

        def radix_body(i, p):
            bit = (31 - i).astype(jnp.uint32)
            t = p | (jnp.uint32(1) << bit)
            hit = jnp.logical_and(xu < t, valid)
            cnt = jnp.sum(hit.astype(jnp.int32), axis=1, keepdims=True)
            return jnp.where(cnt <= K_MEDIAN, t, p)

        p = lax.fori_loop(0, 32, radix_body,
                          jnp.zeros((B_STATIC, 1), jnp.uint32))
        kth = lax.bitcast_convert_type(p, jnp.float32)
        denom = rmax + 1e-8
        norm = deg / denom
        thr = kth / denom
        phi = jnp.where(norm >= thr, jnp.float32(1.0), jnp.float32(VARPHI))
        o_ref[...] = phi[:, :N_NODES]

    return pl.pallas_call(
        body,
        out_shape=jax.ShapeDtypeStruct((B_STATIC, N_NODES), jnp.float32),
    )(partials)


def kernel(edge_index, edge_weights, num_nodes):
    eidx_lin = edge_index.transpose(0, 2, 1).reshape(-1)
    partials = _sc_degree_partials(eidx_lin, edge_weights)
    partials = partials.reshape(B_STATIC, GROUPS_PER_BATCH, N_PAD)
    return _tc_threshold(partials)
